# Initial kernel scaffold; baseline (speedup 1.0000x reference)
import jax
import jax.numpy as jnp
from jax.experimental import pallas as pl


def kernel(x, edge_index, batch, Wp, bp, Wl, att_src, att_dst, bl):
    raise NotImplementedError("write your pallas kernel here")


# speedup vs baseline: 4.3993x; 4.3993x over previous
import functools
import jax
import jax.numpy as jnp
from jax.experimental import pallas as pl

N = 10000
E = 160000
HID = 256
L = 4
H = 8
C = HID // H
G = 400
NEG_SLOPE = 0.2

_BLK = 512
_NPAD = 10240


def _proj_body(x_ref, w_ref, b_ref, o_ref):
    o_ref[...] = (
        jnp.dot(x_ref[...], w_ref[...], preferred_element_type=jnp.float32)
        + b_ref[...]
    )


def _gat_dense_body(h_ref, w_ref, asrc_ref, adst_ref, hp_ref, s_ref, d_ref):
    hp = jnp.dot(h_ref[...], w_ref[...], preferred_element_type=jnp.float32)
    hp_ref[...] = hp
    hph = hp.reshape(hp.shape[0], H, C)
    s_ref[...] = (hph * asrc_ref[...][None, :, :]).sum(-1)
    d_ref[...] = (hph * adst_ref[...][None, :, :]).sum(-1)


def _pad_rows(a):
    return jnp.pad(a, ((0, _NPAD - N), (0, 0)))


@functools.partial(jax.jit, static_argnums=())
def kernel(x, edge_index, batch, Wp, bp, Wl, att_src, att_dst, bl):
    n = x.shape[0]
    grid = _NPAD // _BLK

    xp = _pad_rows(x)
    proj = pl.pallas_call(
        _proj_body,
        grid=(grid,),
        in_specs=[
            pl.BlockSpec((_BLK, HID), lambda i: (i, 0)),
            pl.BlockSpec((HID, HID), lambda i: (0, 0)),
            pl.BlockSpec((1, HID), lambda i: (0, 0)),
        ],
        out_specs=pl.BlockSpec((_BLK, HID), lambda i: (i, 0)),
        out_shape=jax.ShapeDtypeStruct((_NPAD, HID), jnp.float32),
    )
    h = proj(xp, Wp, bp.reshape(1, HID))[:n]

    dense = pl.pallas_call(
        _gat_dense_body,
        grid=(grid,),
        in_specs=[
            pl.BlockSpec((_BLK, HID), lambda i: (i, 0)),
            pl.BlockSpec((HID, HID), lambda i: (0, 0)),
            pl.BlockSpec((H, C), lambda i: (0, 0)),
            pl.BlockSpec((H, C), lambda i: (0, 0)),
        ],
        out_specs=[
            pl.BlockSpec((_BLK, HID), lambda i: (i, 0)),
            pl.BlockSpec((_BLK, H), lambda i: (i, 0)),
            pl.BlockSpec((_BLK, H), lambda i: (i, 0)),
        ],
        out_shape=[
            jax.ShapeDtypeStruct((_NPAD, HID), jnp.float32),
            jax.ShapeDtypeStruct((_NPAD, H), jnp.float32),
            jax.ShapeDtypeStruct((_NPAD, H), jnp.float32),
        ],
    )

    loop = jnp.arange(n, dtype=edge_index.dtype)
    src = jnp.concatenate([edge_index[0], loop])
    dst = jnp.concatenate([edge_index[1], loop])

    for l in range(L):
        hp_pad, a_s, a_d = dense(_pad_rows(h), Wl[l], att_src[l], att_dst[l])
        hp = hp_pad[:n]
        a_s = a_s[:n]
        a_d = a_d[:n]
        e = jax.nn.leaky_relu(a_s[src] + a_d[dst], NEG_SLOPE)
        m = jax.ops.segment_max(e, dst, num_segments=n)
        ex = jnp.exp(e - m[dst])
        den = jax.ops.segment_sum(ex, dst, num_segments=n)
        alpha = ex / (den[dst] + 1e-16)
        msg = (hp[src].reshape(-1, H, C) * alpha[:, :, None]).reshape(-1, HID)
        agg = jax.ops.segment_sum(msg, dst, num_segments=n)
        h = agg + bl[l] + h

    ones = jnp.ones((n,), dtype=h.dtype)
    cnt = jax.ops.segment_sum(ones, batch, num_segments=G)
    mean_p = jax.ops.segment_sum(h, batch, num_segments=G) / jnp.maximum(
        cnt, 1.0
    )[:, None]
    max_p = jax.ops.segment_max(h, batch, num_segments=G)
    return jnp.concatenate([mean_p, max_p], axis=-1)


# speedup vs baseline: 6.5556x; 1.4901x over previous
import functools
import jax
import jax.numpy as jnp
from jax.experimental import pallas as pl

N = 10000
E = 160000
HID = 256
L = 4
H = 8
C = HID // H
G = 400
NEG_SLOPE = 0.2

_BLK = 512
_NPAD = 10240


def _proj_body(x_ref, w_ref, b_ref, o_ref):
    o_ref[...] = (
        jnp.dot(x_ref[...], w_ref[...], preferred_element_type=jnp.float32)
        + b_ref[...]
    )


def _gat_dense_body(h_ref, w_ref, asrc_ref, adst_ref, hp_ref, s_ref, d_ref):
    hp = jnp.dot(h_ref[...], w_ref[...], preferred_element_type=jnp.float32)
    hp_ref[...] = hp
    hph = hp.reshape(hp.shape[0], H, C)
    s_ref[...] = (hph * asrc_ref[...][None, :, :]).sum(-1)
    d_ref[...] = (hph * adst_ref[...][None, :, :]).sum(-1)


def _pad_rows(a):
    return jnp.pad(a, ((0, _NPAD - N), (0, 0)))


@functools.partial(jax.jit, static_argnums=())
def kernel(x, edge_index, batch, Wp, bp, Wl, att_src, att_dst, bl):
    n = x.shape[0]
    grid = _NPAD // _BLK

    xp = _pad_rows(x)
    proj = pl.pallas_call(
        _proj_body,
        grid=(grid,),
        in_specs=[
            pl.BlockSpec((_BLK, HID), lambda i: (i, 0)),
            pl.BlockSpec((HID, HID), lambda i: (0, 0)),
            pl.BlockSpec((1, HID), lambda i: (0, 0)),
        ],
        out_specs=pl.BlockSpec((_BLK, HID), lambda i: (i, 0)),
        out_shape=jax.ShapeDtypeStruct((_NPAD, HID), jnp.float32),
    )
    h = proj(xp, Wp, bp.reshape(1, HID))[:n]

    dense = pl.pallas_call(
        _gat_dense_body,
        grid=(grid,),
        in_specs=[
            pl.BlockSpec((_BLK, HID), lambda i: (i, 0)),
            pl.BlockSpec((HID, HID), lambda i: (0, 0)),
            pl.BlockSpec((H, C), lambda i: (0, 0)),
            pl.BlockSpec((H, C), lambda i: (0, 0)),
        ],
        out_specs=[
            pl.BlockSpec((_BLK, HID), lambda i: (i, 0)),
            pl.BlockSpec((_BLK, H), lambda i: (i, 0)),
            pl.BlockSpec((_BLK, H), lambda i: (i, 0)),
        ],
        out_shape=[
            jax.ShapeDtypeStruct((_NPAD, HID), jnp.float32),
            jax.ShapeDtypeStruct((_NPAD, H), jnp.float32),
            jax.ShapeDtypeStruct((_NPAD, H), jnp.float32),
        ],
    )

    loop = jnp.arange(n, dtype=edge_index.dtype)
    src = jnp.concatenate([edge_index[0], loop])
    dst = jnp.concatenate([edge_index[1], loop])
    perm = jnp.argsort(dst)
    src = src[perm]
    dst = dst[perm]

    for l in range(L):
        hp_pad, a_s, a_d = dense(_pad_rows(h), Wl[l], att_src[l], att_dst[l])
        hp = hp_pad[:n]
        a_s = a_s[:n]
        a_d = a_d[:n]
        e = jax.nn.leaky_relu(a_s[src] + a_d[dst], NEG_SLOPE)
        ex = jnp.exp(e - jnp.max(e, axis=0, keepdims=True))
        den = jax.ops.segment_sum(
            ex, dst, num_segments=n, indices_are_sorted=True
        )
        msg = (hp[src].reshape(-1, H, C) * ex[:, :, None]).reshape(-1, HID)
        agg = jax.ops.segment_sum(
            msg, dst, num_segments=n, indices_are_sorted=True
        )
        agg = agg.reshape(n, H, C) / (den[:, :, None] + 1e-16)
        h = agg.reshape(n, HID) + bl[l] + h

    ones = jnp.ones((n,), dtype=h.dtype)
    cnt = jax.ops.segment_sum(ones, batch, num_segments=G)
    mean_p = jax.ops.segment_sum(h, batch, num_segments=G) / jnp.maximum(
        cnt, 1.0
    )[:, None]
    max_p = jax.ops.segment_max(h, batch, num_segments=G)
    return jnp.concatenate([mean_p, max_p], axis=-1)
